# baseline (speedup 1.0000x reference)
import functools

import jax
import jax.numpy as jnp
from jax import lax
from jax.experimental import pallas as pl
from jax.experimental.pallas import tpu as pltpu
from jax.experimental.pallas import tpu_sc as plsc

_V = 1000
_D = 16
_B = 3_200_000

_info = plsc.get_sparse_core_info()
_NC = _info.num_cores
_NS = _info.num_subcores
_NW = _NC * _NS
_BPW = _B // _NW
_C = 2000
_NCHUNK = _BPW // _C
_UNROLL = 2

_mesh = plsc.VectorSubcoreMesh(core_axis_name="c", subcore_axis_name="s")


@functools.partial(
    pl.kernel,
    mesh=_mesh,
    out_type=jax.ShapeDtypeStruct((_D, _B), jnp.float32),
    scratch_types=[
        pltpu.VMEM((_C,), jnp.int32),
        pltpu.VMEM((_C,), jnp.int32),
        pltpu.VMEM((_D, _C), jnp.float32),
        pltpu.VMEM((_D, _C), jnp.float32),
        pltpu.VMEM((_D, _V), jnp.float32),
        pltpu.SemaphoreType.DMA,
        pltpu.SemaphoreType.DMA,
        pltpu.SemaphoreType.DMA,
        pltpu.SemaphoreType.DMA,
    ],
    compiler_params=pltpu.CompilerParams(
        use_tc_tiling_on_sc=False, needs_layout_passes=False
    ),
)
def _lookup(idx_hbm, tab_t_hbm, out_hbm, idx_a, idx_b, rows_a, rows_b,
            tab_t, si0, si1, so0, so1):
    idx = (idx_a, idx_b)
    rows = (rows_a, rows_b)
    si = (si0, si1)
    so = (so0, so1)
    sid = lax.axis_index("s")
    wid = sid * _NC + lax.axis_index("c")
    base = wid * _BPW

    pltpu.sync_copy(tab_t_hbm, tab_t)

    def istart(g, b):
        pltpu.async_copy(idx_hbm.at[pl.ds(base + g * _C, _C)], idx[b], si[b])

    def iwait(g, b):
        pltpu.make_async_copy(
            idx_hbm.at[pl.ds(base + g * _C, _C)], idx[b], si[b]
        ).wait()

    def ostart(g, b):
        pltpu.async_copy(
            rows[b], out_hbm.at[:, pl.ds(base + g * _C, _C)], so[b]
        )

    def owait(g, b):
        pltpu.make_async_copy(
            rows[b], out_hbm.at[:, pl.ds(base + g * _C, _C)], so[b]
        ).wait()

    cols = [jnp.full((16,), c, jnp.int32) for c in range(_D)]

    def gather_chunk(b):
        @plsc.parallel_loop(0, _C, step=16, unroll=_UNROLL)
        def _(i):
            idxv = idx[b][pl.ds(i, 16)]
            for c in range(_D):
                vals = plsc.load_gather(tab_t, [cols[c], idxv])
                rows[b][c, pl.ds(i, 16)] = vals

    istart(0, 0)
    istart(1, 1)

    def pair(gp, carry):
        g0 = gp * 2

        @pl.when(gp > 0)
        def _():
            owait(g0 - 2, 0)
        iwait(g0, 0)
        gather_chunk(0)
        ostart(g0, 0)

        @pl.when(g0 + 2 < _NCHUNK)
        def _():
            istart(g0 + 2, 0)

        @pl.when(gp > 0)
        def _():
            owait(g0 - 1, 1)
        iwait(g0 + 1, 1)
        gather_chunk(1)
        ostart(g0 + 1, 1)

        @pl.when(g0 + 3 < _NCHUNK)
        def _():
            istart(g0 + 3, 1)
        return carry

    lax.fori_loop(0, _NCHUNK // 2, pair, 0)
    owait(_NCHUNK - 2, 0)
    owait(_NCHUNK - 1, 1)


def kernel(type_indices, type_embedding_weight):
    out_t = _lookup(type_indices, type_embedding_weight.T)
    return out_t.T


# speedup vs baseline: 41.8503x; 41.8503x over previous
import functools

import jax
import jax.numpy as jnp
from jax import lax
from jax.experimental import pallas as pl
from jax.experimental.pallas import tpu as pltpu
from jax.experimental.pallas import tpu_sc as plsc

_V = 1000
_D = 16
_B = 3_200_000

_Q = _B // 128
_TPW = 782
_K = 23
_C = _K * 128
_NCHUNK = _TPW // _K
_RHALF = _K * 8 * 128
_UNROLL = 2

_info = plsc.get_sparse_core_info()
_NC = _info.num_cores
_NS = _info.num_subcores
_NW = _NC * _NS

_mesh = plsc.VectorSubcoreMesh(core_axis_name="c", subcore_axis_name="s")


@functools.partial(
    pl.kernel,
    mesh=_mesh,
    out_type=jax.ShapeDtypeStruct((_B * _D,), jnp.float32),
    scratch_types=[
        pltpu.VMEM((_C,), jnp.int32),
        pltpu.VMEM((_C,), jnp.int32),
        pltpu.VMEM((2 * _RHALF,), jnp.float32),
        pltpu.VMEM((2 * _RHALF,), jnp.float32),
        pltpu.VMEM((_D, _V), jnp.float32),
        pltpu.SemaphoreType.DMA,
        pltpu.SemaphoreType.DMA,
        pltpu.SemaphoreType.DMA,
        pltpu.SemaphoreType.DMA,
    ],
    compiler_params=pltpu.CompilerParams(
        use_tc_tiling_on_sc=False, needs_layout_passes=False
    ),
)
def _lookup(idx_hbm, tab_t_hbm, out_hbm, idx_a, idx_b, rows_a, rows_b,
            tab_t, si0, si1, so0, so1):
    idx = (idx_a, idx_b)
    rows = (rows_a, rows_b)
    si = (si0, si1)
    so = (so0, so1)
    sid = lax.axis_index("s")
    wid = sid * _NC + lax.axis_index("c")
    qs = (wid * (_Q - _TPW)) // (_NW - 1)

    pltpu.sync_copy(tab_t_hbm, tab_t)

    def istart(g, b):
        pltpu.async_copy(
            idx_hbm.at[pl.ds((qs + g * _K) * 128, _C)], idx[b], si[b]
        )

    def iwait(g, b):
        pltpu.make_async_copy(
            idx_hbm.at[pl.ds((qs + g * _K) * 128, _C)], idx[b], si[b]
        ).wait()

    def ostart(g, b):
        qc = qs + g * _K
        for r in range(2):
            pltpu.async_copy(
                rows[b].at[pl.ds(r * _RHALF, _RHALF)],
                out_hbm.at[pl.ds((r * _Q + qc) * 1024, _RHALF)],
                so[b],
            )

    def owait(g, b):
        qc = qs + g * _K
        for r in range(2):
            pltpu.make_async_copy(
                rows[b].at[pl.ds(r * _RHALF, _RHALF)],
                out_hbm.at[pl.ds((r * _Q + qc) * 1024, _RHALF)],
                so[b],
            ).wait()

    cols = [jnp.full((16,), d, jnp.int32) for d in range(_D)]
    dconst = [(d // 8) * _RHALF + (d % 8) * 128 for d in range(_D)]

    def gather_chunk(b):
        @plsc.parallel_loop(0, _C, step=16, unroll=_UNROLL)
        def _(i):
            ib = (i // 128) * 1024 + (i % 128)
            idxv = idx[b][pl.ds(i, 16)]
            for d in range(_D):
                vals = plsc.load_gather(tab_t, [cols[d], idxv])
                rows[b][pl.ds(ib + dconst[d], 16)] = vals

    istart(0, 0)
    istart(1, 1)

    def pair(gp, carry):
        g0 = gp * 2

        @pl.when(gp > 0)
        def _():
            owait(g0 - 2, 0)
        iwait(g0, 0)
        gather_chunk(0)
        ostart(g0, 0)

        @pl.when(g0 + 2 < _NCHUNK)
        def _():
            istart(g0 + 2, 0)

        @pl.when(gp > 0)
        def _():
            owait(g0 - 1, 1)
        iwait(g0 + 1, 1)
        gather_chunk(1)
        ostart(g0 + 1, 1)

        @pl.when(g0 + 3 < _NCHUNK)
        def _():
            istart(g0 + 3, 1)
        return carry

    lax.fori_loop(0, _NCHUNK // 2, pair, 0)
    owait(_NCHUNK - 2, 0)
    owait(_NCHUNK - 1, 1)


def kernel(type_indices, type_embedding_weight):
    flat = _lookup(type_indices, type_embedding_weight.T)
    out4 = flat.reshape(2, _Q, 8, 128)
    return out4.transpose(1, 3, 0, 2).reshape(_B, _D)
